# baseline (speedup 1.0000x reference)
import functools

import jax
import jax.numpy as jnp
from jax.experimental import pallas as pl

_TB = 2048
_SUB = 8

_INV_SQRT2 = 0.7071067811865476


def _gelu_exact(v):
    return 0.5 * v * (1.0 + jax.lax.erf(v * _INV_SQRT2))


def _vqvae_body(nsteps, inv_count,
                x_ref, w1t_ref, b1_ref, w2t_ref, b2_ref, cbt_ref, cbsq_ref,
                cb_ref, w3t_ref, b3_ref, w4t_ref, b4_ref,
                xrec_ref, idx_ref, loss_ref):
    i = pl.program_id(0)
    cs = _TB // _SUB
    parts = []
    for c in range(_SUB):
        rows = pl.ds(c * cs, cs)
        h = _gelu_exact(jnp.dot(x_ref[rows, :], w1t_ref[...],
                                preferred_element_type=jnp.float32) + b1_ref[...])
        z = jnp.dot(h, w2t_ref[...],
                    preferred_element_type=jnp.float32) + b2_ref[...]

        zsq = jnp.sum(z * z, axis=1, keepdims=True)
        zc2 = jnp.dot(2.0 * z, cbt_ref[...], preferred_element_type=jnp.float32)
        dist = (zsq + cbsq_ref[...]) - zc2

        m = jnp.min(dist, axis=1, keepdims=True)
        col = jax.lax.broadcasted_iota(jnp.int32, dist.shape, 1)
        idx = jnp.min(jnp.where(dist <= m, col, dist.shape[1]), axis=1)
        idx_ref[0, 0, rows] = idx

        parts.append(jnp.sum(m))

        onehot = (col == idx[:, None]).astype(jnp.float32)
        z_q = jnp.dot(onehot, cb_ref[...], preferred_element_type=jnp.float32)

        h2 = _gelu_exact(jnp.dot(z_q, w3t_ref[...],
                                 preferred_element_type=jnp.float32) + b3_ref[...])
        xrec_ref[rows, :] = jnp.dot(h2, w4t_ref[...],
                                    preferred_element_type=jnp.float32) + b4_ref[...]

    part = sum(parts).reshape(1, 1)

    @pl.when(i == 0)
    def _():
        loss_ref[...] = jnp.zeros_like(loss_ref)

    loss_ref[...] += part

    @pl.when(i == nsteps - 1)
    def _():
        loss_ref[...] = loss_ref[...] * (1.25 * inv_count)


def kernel(x, W1, b1, W2, b2, codebook, W3, b3, W4, b4):
    B, N, D = x.shape
    T = B * N
    cb_size, cb_dim = codebook.shape
    nsteps = T // _TB

    x2 = x.reshape(T, D)
    cbsq = jnp.sum(codebook * codebook, axis=1).reshape(1, cb_size)

    full = lambda i: (0, 0)
    grid_spec = pl.GridSpec(
        grid=(nsteps,),
        in_specs=[
            pl.BlockSpec((_TB, D), lambda i: (i, 0)),
            pl.BlockSpec((D, W1.shape[0]), full),
            pl.BlockSpec((1, W1.shape[0]), full),
            pl.BlockSpec((W1.shape[0], cb_dim), full),
            pl.BlockSpec((1, cb_dim), full),
            pl.BlockSpec((cb_dim, cb_size), full),
            pl.BlockSpec((1, cb_size), full),
            pl.BlockSpec((cb_size, cb_dim), full),
            pl.BlockSpec((cb_dim, W3.shape[0]), full),
            pl.BlockSpec((1, W3.shape[0]), full),
            pl.BlockSpec((W3.shape[0], D), full),
            pl.BlockSpec((1, D), full),
        ],
        out_specs=[
            pl.BlockSpec((_TB, D), lambda i: (i, 0)),
            pl.BlockSpec((1, 1, _TB), lambda i: (i, 0, 0)),
            pl.BlockSpec((1, 1), full),
        ],
    )
    out_shapes = [
        jax.ShapeDtypeStruct((T, D), jnp.float32),
        jax.ShapeDtypeStruct((nsteps, 1, _TB), jnp.int32),
        jax.ShapeDtypeStruct((1, 1), jnp.float32),
    ]
    body = functools.partial(_vqvae_body, nsteps, 1.0 / (T * cb_dim))
    xrec, idx, loss = pl.pallas_call(
        body,
        grid_spec=grid_spec,
        out_shape=out_shapes,
    )(x2, W1.T, b1.reshape(1, -1), W2.T, b2.reshape(1, -1),
      codebook.T, cbsq, codebook, W3.T, b3.reshape(1, -1),
      W4.T, b4.reshape(1, -1))

    return (xrec.reshape(B, N, D), idx.reshape(B, N), loss.reshape(()))


# speedup vs baseline: 1.3140x; 1.3140x over previous
import functools

import jax
import jax.numpy as jnp
from jax.experimental import pallas as pl

_TB = 3072
_SUB = 6

_INV_SQRT2 = 0.7071067811865476


def _gelu_exact(v):
    return 0.5 * v * (1.0 + jax.lax.erf(v * _INV_SQRT2))


def _vqvae_body(nsteps, inv_count,
                x_ref, w1t_ref, b1_ref, w2t_ref, b2_ref, cbt_ref, cbsq_ref,
                cb_ref, w3t_ref, b3_ref, w4t_ref, b4_ref,
                xrec_ref, idx_ref, loss_ref):
    i = pl.program_id(0)
    cs = _TB // _SUB
    parts = []
    for c in range(_SUB):
        rows = pl.ds(c * cs, cs)
        h = _gelu_exact(jnp.dot(x_ref[rows, :], w1t_ref[...],
                                preferred_element_type=jnp.float32) + b1_ref[...])
        z = jnp.dot(h, w2t_ref[...],
                    preferred_element_type=jnp.float32) + b2_ref[...]

        zsq = jnp.sum(z * z, axis=1, keepdims=True)
        zc2 = jnp.dot(2.0 * z, cbt_ref[...], preferred_element_type=jnp.float32)
        dist = (zsq + cbsq_ref[...]) - zc2

        m = jnp.min(dist, axis=1, keepdims=True)
        col = jax.lax.broadcasted_iota(jnp.int32, dist.shape, 1)
        idx = jnp.min(jnp.where(dist <= m, col, dist.shape[1]), axis=1)
        idx_ref[0, 0, rows] = idx

        parts.append(jnp.sum(m))

        onehot = (col == idx[:, None]).astype(jnp.float32)
        z_q = jnp.dot(onehot, cb_ref[...], preferred_element_type=jnp.float32)

        h2 = _gelu_exact(jnp.dot(z_q, w3t_ref[...],
                                 preferred_element_type=jnp.float32) + b3_ref[...])
        xrec_ref[rows, :] = jnp.dot(h2, w4t_ref[...],
                                    preferred_element_type=jnp.float32) + b4_ref[...]

    part = sum(parts).reshape(1, 1)

    @pl.when(i == 0)
    def _():
        loss_ref[...] = jnp.zeros_like(loss_ref)

    loss_ref[...] += part

    @pl.when(i == nsteps - 1)
    def _():
        loss_ref[...] = loss_ref[...] * (1.25 * inv_count)


def kernel(x, W1, b1, W2, b2, codebook, W3, b3, W4, b4):
    B, N, D = x.shape
    T = B * N
    cb_size, cb_dim = codebook.shape
    nsteps = T // _TB

    x2 = x.reshape(T, D)
    cbsq = jnp.sum(codebook * codebook, axis=1).reshape(1, cb_size)

    full = lambda i: (0, 0)
    grid_spec = pl.GridSpec(
        grid=(nsteps,),
        in_specs=[
            pl.BlockSpec((_TB, D), lambda i: (i, 0)),
            pl.BlockSpec((D, W1.shape[0]), full),
            pl.BlockSpec((1, W1.shape[0]), full),
            pl.BlockSpec((W1.shape[0], cb_dim), full),
            pl.BlockSpec((1, cb_dim), full),
            pl.BlockSpec((cb_dim, cb_size), full),
            pl.BlockSpec((1, cb_size), full),
            pl.BlockSpec((cb_size, cb_dim), full),
            pl.BlockSpec((cb_dim, W3.shape[0]), full),
            pl.BlockSpec((1, W3.shape[0]), full),
            pl.BlockSpec((W3.shape[0], D), full),
            pl.BlockSpec((1, D), full),
        ],
        out_specs=[
            pl.BlockSpec((_TB, D), lambda i: (i, 0)),
            pl.BlockSpec((1, 1, _TB), lambda i: (i, 0, 0)),
            pl.BlockSpec((1, 1), full),
        ],
    )
    out_shapes = [
        jax.ShapeDtypeStruct((T, D), jnp.float32),
        jax.ShapeDtypeStruct((nsteps, 1, _TB), jnp.int32),
        jax.ShapeDtypeStruct((1, 1), jnp.float32),
    ]
    body = functools.partial(_vqvae_body, nsteps, 1.0 / (T * cb_dim))
    xrec, idx, loss = pl.pallas_call(
        body,
        grid_spec=grid_spec,
        out_shape=out_shapes,
    )(x2, W1.T, b1.reshape(1, -1), W2.T, b2.reshape(1, -1),
      codebook.T, cbsq, codebook, W3.T, b3.reshape(1, -1),
      W4.T, b4.reshape(1, -1))

    return (xrec.reshape(B, N, D), idx.reshape(B, N), loss.reshape(()))


# speedup vs baseline: 1.3465x; 1.0247x over previous
import functools

import jax
import jax.numpy as jnp
from jax.experimental import pallas as pl

_TB = 1536
_SUB = 3

_INV_SQRT2 = 0.7071067811865476


def _gelu_exact(v):
    return 0.5 * v * (1.0 + jax.lax.erf(v * _INV_SQRT2))


def _vqvae_body(nsteps, inv_count,
                x_ref, w1t_ref, b1_ref, w2t_ref, b2_ref, cbt_ref, cbsq_ref,
                cb_ref, w3t_ref, b3_ref, w4t_ref, b4_ref,
                xrec_ref, idx_ref, loss_ref):
    i = pl.program_id(0)
    cs = _TB // _SUB
    parts = []
    for c in range(_SUB):
        rows = pl.ds(c * cs, cs)
        h = _gelu_exact(jnp.dot(x_ref[rows, :], w1t_ref[...],
                                preferred_element_type=jnp.float32) + b1_ref[...])
        z = jnp.dot(h, w2t_ref[...],
                    preferred_element_type=jnp.float32) + b2_ref[...]

        zsq = jnp.sum(z * z, axis=1, keepdims=True)
        zc2 = jnp.dot(2.0 * z, cbt_ref[...], preferred_element_type=jnp.float32)
        dist = (zsq + cbsq_ref[...]) - zc2

        m = jnp.min(dist, axis=1, keepdims=True)
        col = jax.lax.broadcasted_iota(jnp.int32, dist.shape, 1)
        idx = jnp.min(jnp.where(dist <= m, col, dist.shape[1]), axis=1)
        idx_ref[0, 0, rows] = idx

        parts.append(jnp.sum(m))

        onehot = (col == idx[:, None]).astype(jnp.float32)
        z_q = jnp.dot(onehot, cb_ref[...], preferred_element_type=jnp.float32)

        h2 = _gelu_exact(jnp.dot(z_q, w3t_ref[...],
                                 preferred_element_type=jnp.float32) + b3_ref[...])
        xrec_ref[rows, :] = jnp.dot(h2, w4t_ref[...],
                                    preferred_element_type=jnp.float32) + b4_ref[...]

    part = sum(parts).reshape(1, 1)

    @pl.when(i == 0)
    def _():
        loss_ref[...] = jnp.zeros_like(loss_ref)

    loss_ref[...] += part

    @pl.when(i == nsteps - 1)
    def _():
        loss_ref[...] = loss_ref[...] * (1.25 * inv_count)


def kernel(x, W1, b1, W2, b2, codebook, W3, b3, W4, b4):
    B, N, D = x.shape
    T = B * N
    cb_size, cb_dim = codebook.shape
    nsteps = T // _TB

    x2 = x.reshape(T, D)
    cbsq = jnp.sum(codebook * codebook, axis=1).reshape(1, cb_size)

    full = lambda i: (0, 0)
    grid_spec = pl.GridSpec(
        grid=(nsteps,),
        in_specs=[
            pl.BlockSpec((_TB, D), lambda i: (i, 0)),
            pl.BlockSpec((D, W1.shape[0]), full),
            pl.BlockSpec((1, W1.shape[0]), full),
            pl.BlockSpec((W1.shape[0], cb_dim), full),
            pl.BlockSpec((1, cb_dim), full),
            pl.BlockSpec((cb_dim, cb_size), full),
            pl.BlockSpec((1, cb_size), full),
            pl.BlockSpec((cb_size, cb_dim), full),
            pl.BlockSpec((cb_dim, W3.shape[0]), full),
            pl.BlockSpec((1, W3.shape[0]), full),
            pl.BlockSpec((W3.shape[0], D), full),
            pl.BlockSpec((1, D), full),
        ],
        out_specs=[
            pl.BlockSpec((_TB, D), lambda i: (i, 0)),
            pl.BlockSpec((1, 1, _TB), lambda i: (i, 0, 0)),
            pl.BlockSpec((1, 1), full),
        ],
    )
    out_shapes = [
        jax.ShapeDtypeStruct((T, D), jnp.float32),
        jax.ShapeDtypeStruct((nsteps, 1, _TB), jnp.int32),
        jax.ShapeDtypeStruct((1, 1), jnp.float32),
    ]
    body = functools.partial(_vqvae_body, nsteps, 1.0 / (T * cb_dim))
    xrec, idx, loss = pl.pallas_call(
        body,
        grid_spec=grid_spec,
        out_shape=out_shapes,
    )(x2, W1.T, b1.reshape(1, -1), W2.T, b2.reshape(1, -1),
      codebook.T, cbsq, codebook, W3.T, b3.reshape(1, -1),
      W4.T, b4.reshape(1, -1))

    return (xrec.reshape(B, N, D), idx.reshape(B, N), loss.reshape(()))
